# baseline (speedup 1.0000x reference)
import jax
import jax.numpy as jnp
from jax.experimental import pallas as pl

N = 10000
NB = 64


def _layer_norm(x, g, b):
    mu = jnp.mean(x, axis=-1, keepdims=True)
    var = jnp.mean((x - mu) ** 2, axis=-1, keepdims=True)
    return (x - mu) / jnp.sqrt(var + 1e-5) * g + b


def _genconv(h, src, dst, ea, t, W1, b1, lg, lb, W2, b2):
    msg = jax.nn.relu(h[src] + ea) + 1e-7
    scores = msg * t
    smax = jax.ops.segment_max(scores, dst, num_segments=N)
    ex = jnp.exp(scores - smax[dst])
    den = jax.ops.segment_sum(ex, dst, num_segments=N)
    alpha = ex / (den[dst] + 1e-16)
    agg = jax.ops.segment_sum(msg * alpha, dst, num_segments=N)
    z = (h + agg) @ W1 + b1
    z = _layer_norm(z, lg, lb)
    z = jax.nn.relu(z)
    return z @ W2 + b2


def _copy_k(x_ref, o_ref):
    o_ref[...] = x_ref[...]


def kernel(x, edge_index, edge_attr, batch, node_W, node_b, edge_W, edge_b, t, W1, b1, ln_g, ln_b, W2, b2, norm_g, norm_b, lin_W, lin_b):
    src, dst = edge_index[0], edge_index[1]
    perm = jnp.argsort(dst)
    src = src[perm]
    dst = dst[perm]
    edge_attr = edge_attr[perm]
    h = x @ node_W + node_b
    h = pl.pallas_call(
        _copy_k, out_shape=jax.ShapeDtypeStruct(h.shape, h.dtype))(h)
    ea = edge_attr @ edge_W + edge_b
    h = _genconv(h, src, dst, ea, t[0], W1[0], b1[0], ln_g[0], ln_b[0], W2[0], b2[0])
    for l in range(1, 3):
        r = h
        z = jax.nn.relu(_layer_norm(h, norm_g[l], norm_b[l]))
        h = _genconv(z, src, dst, ea, t[l], W1[l], b1[l], ln_g[l], ln_b[l], W2[l], b2[l]) + r
    h = jax.nn.relu(_layer_norm(h, norm_g[0], norm_b[0]))
    emb = h
    sv = emb @ lin_W + lin_b
    svb = jax.ops.segment_sum(sv, batch, num_segments=NB).T
    ge = jax.ops.segment_sum(emb, batch, num_segments=NB)
    return (svb, ge, emb)


# speedup vs baseline: 3.9972x; 3.9972x over previous
import functools

import jax
import jax.numpy as jnp
from jax import lax
from jax.experimental import pallas as pl
from jax.experimental.pallas import tpu as pltpu
from jax.experimental.pallas import tpu_sc as plsc

N = 10000
E = 320000
H = 128
L = 3
NB = 64
NW = 32
C = 128
NEG = -1e30
KV = H // 16


def _sc_agg_body(h_hbm, ea_hbm, src_hbm, dst_hbm, meta_hbm, tv_hbm, out_hbm,
                 meta_v, tv_v, idx_v, dst_v, ea_v, hs_v, row_v, zrow_v, sem):
    wid = lax.axis_index("s") * 2 + lax.axis_index("c")
    pltpu.sync_copy(meta_hbm.at[wid], meta_v)
    pltpu.sync_copy(tv_hbm, tv_v)
    mrow = meta_v[...]
    f0 = mrow[0]
    f1 = mrow[1]
    g0 = mrow[2]
    g1 = mrow[3]
    for k in range(KV):
        zrow_v[pl.ds(k * 16, 16)] = jnp.zeros((16,), jnp.float32)

    def zero_rows(lo, hi):
        def zbody(r, _):
            pltpu.sync_copy(zrow_v, out_hbm.at[r])
            return 0
        lax.fori_loop(lo, hi, zbody, 0)

    tvec = tv_v[...]
    negv = jnp.full((16,), NEG, jnp.float32)

    ca0 = (f0 // C) * C
    nch = jnp.maximum(0, (f1 - ca0 + (C - 1)) // C)

    def chunk_body(i, carry):
        ca = ca0 + i * C
        pltpu.sync_copy(src_hbm.at[pl.ds(ca, C)], idx_v)
        pltpu.sync_copy(dst_hbm.at[pl.ds(ca, C)], dst_v.at[pl.ds(0, C)])
        pltpu.sync_copy(ea_hbm.at[pl.ds(ca, C), :], ea_v)
        pltpu.async_copy(h_hbm.at[idx_v], hs_v, sem).wait()

        def edge_body(j, car):
            d_cur = car[0]
            ms = car[1:1 + KV]
            ss = car[1 + KV:1 + 2 * KV]
            aa = car[1 + 2 * KV:1 + 3 * KV]
            e = ca + j
            d = dst_v[pl.ds(j, 16)][0]
            valid = (e >= f0) & (e < f1)
            is_new = valid & (d != d_cur)
            fin = is_new & (d_cur >= 0)

            @pl.when(fin)
            def _():
                for k in range(KV):
                    row_v[pl.ds(k * 16, 16)] = aa[k] / (ss[k] + 1e-16)
                pltpu.sync_copy(row_v, out_hbm.at[d_cur])

            @pl.when(is_new)
            def _():
                zlo = jnp.where(d_cur >= 0, d_cur + 1, g0)
                zero_rows(zlo, d)

            nm, ns, na = [], [], []
            for k in range(KV):
                hrow = hs_v[j, pl.ds(k * 16, 16)]
                earow = ea_v[j, pl.ds(k * 16, 16)]
                msg = jnp.maximum(hrow + earow, 0.0) + 1e-7
                sc = msg * tvec
                mk = jnp.where(is_new, negv, ms[k])
                mn = jnp.maximum(mk, sc)
                corr = jnp.exp(mk - mn)
                e1 = jnp.exp(sc - mn)
                s_u = ss[k] * corr + e1
                a_u = aa[k] * corr + msg * e1
                nm.append(jnp.where(valid, mn, ms[k]))
                ns.append(jnp.where(valid, s_u, ss[k]))
                na.append(jnp.where(valid, a_u, aa[k]))
            d_n = jnp.where(is_new, d, d_cur)
            return (d_n,) + tuple(nm) + tuple(ns) + tuple(na)

        return lax.fori_loop(0, C, edge_body, carry)

    zeros16 = jnp.zeros((16,), jnp.float32)
    carry0 = ((jnp.int32(-1),) + (negv,) * KV + (zeros16,) * KV
              + (zeros16,) * KV)
    carry = lax.fori_loop(0, nch, chunk_body, carry0)
    d_cur = carry[0]
    ss = carry[1 + KV:1 + 2 * KV]
    aa = carry[1 + 2 * KV:1 + 3 * KV]

    @pl.when(d_cur >= 0)
    def _():
        for k in range(KV):
            row_v[pl.ds(k * 16, 16)] = aa[k] / (ss[k] + 1e-16)
        pltpu.sync_copy(row_v, out_hbm.at[d_cur])

    zlo = jnp.where(d_cur >= 0, d_cur + 1, g0)
    zero_rows(zlo, g1)


_sc_agg = functools.partial(
    pl.kernel,
    out_type=jax.ShapeDtypeStruct((N, H), jnp.float32),
    mesh=plsc.VectorSubcoreMesh(core_axis_name="c", subcore_axis_name="s"),
    scratch_types=[
        pltpu.VMEM((16,), jnp.int32),
        pltpu.VMEM((16,), jnp.float32),
        pltpu.VMEM((C,), jnp.int32),
        pltpu.VMEM((C + 16,), jnp.int32),
        pltpu.VMEM((C, H), jnp.float32),
        pltpu.VMEM((C, H), jnp.float32),
        pltpu.VMEM((H,), jnp.float32),
        pltpu.VMEM((H,), jnp.float32),
        pltpu.SemaphoreType.DMA,
    ],
)(_sc_agg_body)


def _enc_body(x_ref, w_ref, b_ref, o_ref):
    o_ref[...] = (
        jnp.dot(x_ref[...], w_ref[...], preferred_element_type=jnp.float32, precision=lax.Precision.HIGHEST)
        + b_ref[...])


def _mlp_body(*refs, add_res):
    if add_res:
        (h_ref, a_ref, res_ref, w1_ref, b1_ref, lg_ref, lb_ref, w2_ref,
         b2_ref, o_ref) = refs
    else:
        (h_ref, a_ref, w1_ref, b1_ref, lg_ref, lb_ref, w2_ref, b2_ref,
         o_ref) = refs
        res_ref = None
    z = h_ref[...] + a_ref[...]
    z1 = jnp.dot(z, w1_ref[...], preferred_element_type=jnp.float32, precision=lax.Precision.HIGHEST) + b1_ref[...]
    mu = jnp.mean(z1, axis=-1, keepdims=True)
    var = jnp.mean((z1 - mu) ** 2, axis=-1, keepdims=True)
    z1 = (z1 - mu) / jnp.sqrt(var + 1e-5) * lg_ref[...] + lb_ref[...]
    z1 = jnp.maximum(z1, 0.0)
    out = jnp.dot(z1, w2_ref[...], preferred_element_type=jnp.float32, precision=lax.Precision.HIGHEST) + b2_ref[...]
    if res_ref is not None:
        out = out + res_ref[...]
    o_ref[...] = out


def _prenorm_body(h_ref, g_ref, b_ref, o_ref):
    h = h_ref[...]
    mu = jnp.mean(h, axis=-1, keepdims=True)
    var = jnp.mean((h - mu) ** 2, axis=-1, keepdims=True)
    o_ref[...] = jnp.maximum((h - mu) / jnp.sqrt(var + 1e-5) * g_ref[...]
                             + b_ref[...], 0.0)


def _final_body(h_ref, g_ref, b_ref, bat_ref, lw_ref, lb_ref,
                emb_ref, ge_ref, svb_ref, ge_acc, cnt_acc):
    pid = pl.program_id(0)
    h = h_ref[...]
    mu = jnp.mean(h, axis=-1, keepdims=True)
    var = jnp.mean((h - mu) ** 2, axis=-1, keepdims=True)
    emb = jnp.maximum((h - mu) / jnp.sqrt(var + 1e-5) * g_ref[...]
                      + b_ref[...], 0.0)
    emb_ref[...] = emb
    r = emb.shape[0]
    oh = (bat_ref[...] == lax.broadcasted_iota(jnp.int32, (r, NB), 1)
          ).astype(jnp.float32)

    @pl.when(pid == 0)
    def _():
        ge_acc[...] = jnp.zeros_like(ge_acc)
        cnt_acc[...] = jnp.zeros_like(cnt_acc)

    ge_acc[...] += lax.dot_general(oh, emb, (((0,), (0,)), ((), ())),
                                   preferred_element_type=jnp.float32,
                                   precision=lax.Precision.HIGHEST)
    cnt_acc[...] += jnp.sum(oh, axis=0, keepdims=True)

    @pl.when(pid == pl.num_programs(0) - 1)
    def _():
        ge_ref[...] = ge_acc[...]
        svb = lax.dot_general(lw_ref[...], ge_acc[...],
                              (((0,), (1,)), ((), ())),
                              preferred_element_type=jnp.float32,
                              precision=lax.Precision.HIGHEST)
        svb_ref[...] = svb + cnt_acc[...] * lb_ref[...]


_RB = 2000


def _tc_encode(x, w, b):
    n, din = x.shape
    dout = w.shape[1]
    rb = _RB if n % _RB == 0 else 8000
    return pl.pallas_call(
        _enc_body,
        grid=(n // rb,),
        in_specs=[
            pl.BlockSpec((rb, din), lambda i: (i, 0)),
            pl.BlockSpec((din, dout), lambda i: (0, 0)),
            pl.BlockSpec((1, dout), lambda i: (0, 0)),
        ],
        out_specs=pl.BlockSpec((rb, dout), lambda i: (i, 0)),
        out_shape=jax.ShapeDtypeStruct((n, dout), jnp.float32),
    )(x, w, b.reshape(1, dout))


def _tc_mlp(h, agg, res, w1, b1, lg, lb, w2, b2):
    add_res = res is not None
    row_specs = [pl.BlockSpec((_RB, H), lambda i: (i, 0))] * (3 if add_res else 2)
    args = ([h, agg, res] if add_res else [h, agg]) + [
        w1, b1.reshape(1, -1), lg.reshape(1, -1), lb.reshape(1, -1),
        w2, b2.reshape(1, -1)]
    return pl.pallas_call(
        functools.partial(_mlp_body, add_res=add_res),
        grid=(N // _RB,),
        in_specs=row_specs + [
            pl.BlockSpec((H, 2 * H), lambda i: (0, 0)),
            pl.BlockSpec((1, 2 * H), lambda i: (0, 0)),
            pl.BlockSpec((1, 2 * H), lambda i: (0, 0)),
            pl.BlockSpec((1, 2 * H), lambda i: (0, 0)),
            pl.BlockSpec((2 * H, H), lambda i: (0, 0)),
            pl.BlockSpec((1, H), lambda i: (0, 0)),
        ],
        out_specs=pl.BlockSpec((_RB, H), lambda i: (i, 0)),
        out_shape=jax.ShapeDtypeStruct((N, H), jnp.float32),
    )(*args)


def _tc_prenorm(h, g, b):
    return pl.pallas_call(
        _prenorm_body,
        grid=(N // _RB,),
        in_specs=[
            pl.BlockSpec((_RB, H), lambda i: (i, 0)),
            pl.BlockSpec((1, H), lambda i: (0, 0)),
            pl.BlockSpec((1, H), lambda i: (0, 0)),
        ],
        out_specs=pl.BlockSpec((_RB, H), lambda i: (i, 0)),
        out_shape=jax.ShapeDtypeStruct((N, H), jnp.float32),
    )(h, g.reshape(1, H), b.reshape(1, H))


def _tc_final(h, g, b, batch2d, lw, lb):
    return pl.pallas_call(
        _final_body,
        grid=(N // _RB,),
        in_specs=[
            pl.BlockSpec((_RB, H), lambda i: (i, 0)),
            pl.BlockSpec((1, H), lambda i: (0, 0)),
            pl.BlockSpec((1, H), lambda i: (0, 0)),
            pl.BlockSpec((_RB, 1), lambda i: (i, 0)),
            pl.BlockSpec((H, 1), lambda i: (0, 0)),
            pl.BlockSpec((1, 1), lambda i: (0, 0)),
        ],
        out_specs=[
            pl.BlockSpec((_RB, H), lambda i: (i, 0)),
            pl.BlockSpec((NB, H), lambda i: (0, 0)),
            pl.BlockSpec((1, NB), lambda i: (0, 0)),
        ],
        out_shape=[
            jax.ShapeDtypeStruct((N, H), jnp.float32),
            jax.ShapeDtypeStruct((NB, H), jnp.float32),
            jax.ShapeDtypeStruct((1, NB), jnp.float32),
        ],
        scratch_shapes=[
            pltpu.VMEM((NB, H), jnp.float32),
            pltpu.VMEM((1, NB), jnp.float32),
        ],
    )(h, g.reshape(1, H), b.reshape(1, H), batch2d, lw, lb.reshape(1, 1))


def kernel(x, edge_index, edge_attr, batch, node_W, node_b, edge_W, edge_b,
           t, W1, b1, ln_g, ln_b, W2, b2, norm_g, norm_b, lin_W, lin_b):
    src, dst = edge_index[0], edge_index[1]
    perm = jnp.argsort(dst)
    dst_s = dst[perm]
    src_s = src[perm]
    ea_attr_s = edge_attr[perm]

    base = jnp.arange(1, NW, dtype=jnp.int32) * (E // NW)
    fmid = jnp.searchsorted(dst_s, dst_s[base - 1], side="right").astype(jnp.int32)
    f = jnp.concatenate([jnp.zeros((1,), jnp.int32), fmid,
                         jnp.full((1,), E, jnp.int32)])
    gmid = jnp.where(fmid < E, dst_s[jnp.minimum(fmid, E - 1)], N)
    g = jnp.concatenate([jnp.zeros((1,), jnp.int32), gmid,
                         jnp.full((1,), N, jnp.int32)])
    meta = jnp.zeros((NW, 16), jnp.int32)
    meta = meta.at[:, 0].set(f[:-1]).at[:, 1].set(f[1:])
    meta = meta.at[:, 2].set(g[:-1]).at[:, 3].set(g[1:])

    h = _tc_encode(x, node_W, node_b)
    ea = _tc_encode(ea_attr_s, edge_W, edge_b)

    for l in range(L):
        z = h if l == 0 else _tc_prenorm(h, norm_g[l], norm_b[l])
        tv = jnp.full((16,), t[l], jnp.float32)
        agg = _sc_agg(z, ea, src_s, dst_s, meta, tv)
        h = _tc_mlp(z, agg, None if l == 0 else h,
                    W1[l], b1[l], ln_g[l], ln_b[l], W2[l], b2[l])

    emb, ge, svb = _tc_final(h, norm_g[0], norm_b[0],
                             batch.reshape(N, 1), lin_W, lin_b)
    return (svb, ge, emb)


# speedup vs baseline: 6.2632x; 1.5669x over previous
import functools

import jax
import jax.numpy as jnp
from jax import lax
from jax.experimental import pallas as pl
from jax.experimental.pallas import tpu as pltpu
from jax.experimental.pallas import tpu_sc as plsc

N = 10000
E = 320000
H = 128
L = 3
NB = 64
NW = 32
C = 128
HC = C // 2
NEG = -1e30
KV = H // 16


def _sc_agg_body(h_hbm, ea_hbm, src_hbm, dst_hbm, perm_hbm, meta_hbm, tv_hbm,
                 out_hbm, meta_v, tv_v, idx_v0, idx_v1, prm_v0, prm_v1,
                 dst_v0, dst_v1, ea_v0, ea_v1, hs_v0, hs_v1, row_v0, row_v1,
                 zrow_v, sidx0, sidx1, soth0, soth1, sgat0, sgat1, sw0, sw1):
    wid = lax.axis_index("s") * 2 + lax.axis_index("c")
    pltpu.sync_copy(meta_hbm.at[wid], meta_v)
    pltpu.sync_copy(tv_hbm, tv_v)
    mrow = meta_v[...]
    f0 = mrow[0]
    f1 = mrow[1]
    g0 = mrow[2]
    g1 = mrow[3]
    for k in range(KV):
        zrow_v[pl.ds(k * 16, 16)] = jnp.zeros((16,), jnp.float32)

    def zero_rows(lo, hi):
        def zbody(r, _):
            pltpu.sync_copy(zrow_v, out_hbm.at[r])
            return 0
        lax.fori_loop(lo, hi, zbody, 0)

    tvec = tv_v[...]
    negv = jnp.full((16,), NEG, jnp.float32)
    iota16 = lax.iota(jnp.int32, 16)

    ca0 = (f0 // C) * C
    nch = jnp.maximum(0, (f1 - ca0 + (C - 1)) // C)

    bufs = ((idx_v0, prm_v0, dst_v0, ea_v0, hs_v0, sidx0, soth0, sgat0),
            (idx_v1, prm_v1, dst_v1, ea_v1, hs_v1, sidx1, soth1, sgat1))

    def issue_in(c, p):
        idx_v, prm_v, dst_v, ea_v, hs_v, sidx, soth, sgat = bufs[p]

        @pl.when(c < nch)
        def _():
            ca = ca0 + c * C
            pltpu.async_copy(src_hbm.at[pl.ds(ca, C)], idx_v, sidx)
            pltpu.async_copy(perm_hbm.at[pl.ds(ca, C)], prm_v, sidx)
            pltpu.async_copy(dst_hbm.at[pl.ds(ca, C)],
                             dst_v.at[pl.ds(0, C)], soth)

    def wait_idx(c, p):
        idx_v, prm_v, dst_v, ea_v, hs_v, sidx, soth, sgat = bufs[p]

        @pl.when(c < nch)
        def _():
            pltpu.make_async_copy(src_hbm.at[pl.ds(0, C)], idx_v, sidx).wait()
            pltpu.make_async_copy(perm_hbm.at[pl.ds(0, C)], prm_v, sidx).wait()

    def issue_gather(c, p):
        idx_v, prm_v, dst_v, ea_v, hs_v, sidx, soth, sgat = bufs[p]

        @pl.when(c < nch)
        def _():
            pltpu.async_copy(h_hbm.at[idx_v], hs_v, sgat)
            pltpu.async_copy(ea_hbm.at[prm_v], ea_v, sgat)

    def wait_chunk(c, p):
        idx_v, prm_v, dst_v, ea_v, hs_v, sidx, soth, sgat = bufs[p]

        @pl.when(c < nch)
        def _():
            pltpu.make_async_copy(dst_hbm.at[pl.ds(0, C)],
                                  dst_v.at[pl.ds(0, C)], soth).wait()
            pltpu.make_async_copy(h_hbm.at[idx_v], hs_v, sgat).wait()
            pltpu.make_async_copy(ea_hbm.at[prm_v], ea_v, sgat).wait()

    def emit_row(seg_cnt, d_cur, ss, aa):
        for par, (row_v, sw) in enumerate(((row_v0, sw0), (row_v1, sw1))):
            @pl.when(lax.rem(seg_cnt, 2) == par)
            def _():
                @pl.when(seg_cnt >= 2)
                def _():
                    pltpu.make_async_copy(row_v, out_hbm.at[0], sw).wait()
                for k in range(KV):
                    row_v[pl.ds(k * 16, 16)] = aa[k] / (ss[k] + 1e-16)
                pltpu.async_copy(row_v, out_hbm.at[d_cur], sw)

    def half_compute(c, p, half, carry):
        idx_v, prm_v, dst_v, ea_v, hs_v, sidx, soth, sgat = bufs[p]
        ca = ca0 + c * C
        jlo = jnp.clip(f0 - ca, half * HC, half * HC + HC)
        jhi = jnp.clip(f1 - ca, half * HC, half * HC + HC)

        def edge_body(j, car):
            d_cur = car[0]
            cnt = car[1]
            ms = car[2:2 + KV]
            ss = car[2 + KV:2 + 2 * KV]
            aa = car[2 + 2 * KV:2 + 3 * KV]
            d = dst_v[pl.ds(j, 16)][0]
            is_new = d != d_cur
            fin = is_new & (d_cur >= 0)

            @pl.when(fin)
            def _():
                emit_row(cnt, d_cur, ss, aa)

            @pl.when(is_new)
            def _():
                zlo = jnp.where(d_cur >= 0, d_cur + 1, g0)
                zero_rows(zlo, d)

            cnt_n = jnp.where(fin, cnt + 1, cnt)
            nm, ns, na = [], [], []
            for k in range(KV):
                hrow = hs_v[j, pl.ds(k * 16, 16)]
                earow = ea_v[j, pl.ds(k * 16, 16)]
                msg = jnp.maximum(hrow + earow, 0.0) + 1e-7
                sc = msg * tvec
                mk = jnp.where(is_new, negv, ms[k])
                mn = jnp.maximum(mk, sc)
                corr = jnp.exp(mk - mn)
                e1 = jnp.exp(sc - mn)
                nm.append(mn)
                ns.append(ss[k] * corr + e1)
                na.append(aa[k] * corr + msg * e1)
            d_n = jnp.where(is_new, d, d_cur)
            return (d_n, cnt_n) + tuple(nm) + tuple(ns) + tuple(na)

        return lax.fori_loop(jlo, jhi, edge_body, carry)

    issue_in(0, 0)
    issue_in(1, 1)
    wait_idx(0, 0)
    issue_gather(0, 0)

    zeros16 = jnp.zeros((16,), jnp.float32)
    carry0 = ((jnp.int32(-1), jnp.int32(0))
              + (negv,) * KV + (zeros16,) * KV + (zeros16,) * KV)

    def pair_body(g, carry):
        c_a = 2 * g
        c_b = c_a + 1
        wait_chunk(c_a, 0)
        carry = half_compute(c_a, 0, 0, carry)
        wait_idx(c_b, 1)
        issue_gather(c_b, 1)
        carry = half_compute(c_a, 0, 1, carry)
        issue_in(c_a + 2, 0)
        wait_chunk(c_b, 1)
        carry = half_compute(c_b, 1, 0, carry)
        wait_idx(c_a + 2, 0)
        issue_gather(c_a + 2, 0)
        carry = half_compute(c_b, 1, 1, carry)
        issue_in(c_b + 2, 1)
        return carry

    npair = (nch + 1) // 2
    carry = lax.fori_loop(0, npair, pair_body, carry0)
    d_cur = carry[0]
    cnt = carry[1]
    ss = carry[2 + KV:2 + 2 * KV]
    aa = carry[2 + 2 * KV:2 + 3 * KV]

    @pl.when(d_cur >= 0)
    def _():
        emit_row(cnt, d_cur, ss, aa)

    cnt = jnp.where(d_cur >= 0, cnt + 1, cnt)

    @pl.when(cnt >= 1)
    def _():
        pltpu.make_async_copy(row_v0, out_hbm.at[0], sw0).wait()

    @pl.when(cnt >= 2)
    def _():
        pltpu.make_async_copy(row_v1, out_hbm.at[0], sw1).wait()

    zlo = jnp.where(d_cur >= 0, d_cur + 1, g0)
    zero_rows(zlo, g1)


_sc_agg = functools.partial(
    pl.kernel,
    out_type=jax.ShapeDtypeStruct((N + 16, H), jnp.float32),
    mesh=plsc.VectorSubcoreMesh(core_axis_name="c", subcore_axis_name="s"),
    scratch_types=[
        pltpu.VMEM((16,), jnp.int32),
        pltpu.VMEM((16,), jnp.float32),
        pltpu.VMEM((C,), jnp.int32),
        pltpu.VMEM((C,), jnp.int32),
        pltpu.VMEM((C,), jnp.int32),
        pltpu.VMEM((C,), jnp.int32),
        pltpu.VMEM((C + 16,), jnp.int32),
        pltpu.VMEM((C + 16,), jnp.int32),
        pltpu.VMEM((C, H), jnp.float32),
        pltpu.VMEM((C, H), jnp.float32),
        pltpu.VMEM((C, H), jnp.float32),
        pltpu.VMEM((C, H), jnp.float32),
        pltpu.VMEM((H,), jnp.float32),
        pltpu.VMEM((H,), jnp.float32),
        pltpu.VMEM((H,), jnp.float32),
        pltpu.SemaphoreType.DMA,
        pltpu.SemaphoreType.DMA,
        pltpu.SemaphoreType.DMA,
        pltpu.SemaphoreType.DMA,
        pltpu.SemaphoreType.DMA,
        pltpu.SemaphoreType.DMA,
        pltpu.SemaphoreType.DMA,
        pltpu.SemaphoreType.DMA,
    ],
)(_sc_agg_body)


def _enc_body(x_ref, w_ref, b_ref, o_ref):
    o_ref[...] = (
        jnp.dot(x_ref[...], w_ref[...], preferred_element_type=jnp.float32, precision=lax.Precision.HIGHEST)
        + b_ref[...])


def _mlp_body(*refs, add_res):
    if add_res:
        (h_ref, a_ref, res_ref, w1_ref, b1_ref, lg_ref, lb_ref, w2_ref,
         b2_ref, o_ref) = refs
    else:
        (h_ref, a_ref, w1_ref, b1_ref, lg_ref, lb_ref, w2_ref, b2_ref,
         o_ref) = refs
        res_ref = None
    z = h_ref[...] + a_ref[...]
    z1 = jnp.dot(z, w1_ref[...], preferred_element_type=jnp.float32, precision=lax.Precision.HIGHEST) + b1_ref[...]
    mu = jnp.mean(z1, axis=-1, keepdims=True)
    var = jnp.mean((z1 - mu) ** 2, axis=-1, keepdims=True)
    z1 = (z1 - mu) / jnp.sqrt(var + 1e-5) * lg_ref[...] + lb_ref[...]
    z1 = jnp.maximum(z1, 0.0)
    out = jnp.dot(z1, w2_ref[...], preferred_element_type=jnp.float32, precision=lax.Precision.HIGHEST) + b2_ref[...]
    if res_ref is not None:
        out = out + res_ref[...]
    o_ref[...] = out


def _prenorm_body(h_ref, g_ref, b_ref, o_ref):
    h = h_ref[...]
    mu = jnp.mean(h, axis=-1, keepdims=True)
    var = jnp.mean((h - mu) ** 2, axis=-1, keepdims=True)
    o_ref[...] = jnp.maximum((h - mu) / jnp.sqrt(var + 1e-5) * g_ref[...]
                             + b_ref[...], 0.0)


def _final_body(h_ref, g_ref, b_ref, bat_ref, lw_ref, lb_ref,
                emb_ref, ge_ref, svb_ref, ge_acc, cnt_acc):
    pid = pl.program_id(0)
    h = h_ref[...]
    mu = jnp.mean(h, axis=-1, keepdims=True)
    var = jnp.mean((h - mu) ** 2, axis=-1, keepdims=True)
    emb = jnp.maximum((h - mu) / jnp.sqrt(var + 1e-5) * g_ref[...]
                      + b_ref[...], 0.0)
    emb_ref[...] = emb
    r = emb.shape[0]
    oh = (bat_ref[...] == lax.broadcasted_iota(jnp.int32, (r, NB), 1)
          ).astype(jnp.float32)

    @pl.when(pid == 0)
    def _():
        ge_acc[...] = jnp.zeros_like(ge_acc)
        cnt_acc[...] = jnp.zeros_like(cnt_acc)

    ge_acc[...] += lax.dot_general(oh, emb, (((0,), (0,)), ((), ())),
                                   preferred_element_type=jnp.float32,
                                   precision=lax.Precision.HIGHEST)
    cnt_acc[...] += jnp.sum(oh, axis=0, keepdims=True)

    @pl.when(pid == pl.num_programs(0) - 1)
    def _():
        ge_ref[...] = ge_acc[...]
        svb = lax.dot_general(lw_ref[...], ge_acc[...],
                              (((0,), (1,)), ((), ())),
                              preferred_element_type=jnp.float32,
                              precision=lax.Precision.HIGHEST)
        svb_ref[...] = svb + cnt_acc[...] * lb_ref[...]


_RB = 2000


def _tc_encode(x, w, b):
    n, din = x.shape
    dout = w.shape[1]
    rb = _RB if n % _RB == 0 else 8000
    return pl.pallas_call(
        _enc_body,
        grid=(n // rb,),
        in_specs=[
            pl.BlockSpec((rb, din), lambda i: (i, 0)),
            pl.BlockSpec((din, dout), lambda i: (0, 0)),
            pl.BlockSpec((1, dout), lambda i: (0, 0)),
        ],
        out_specs=pl.BlockSpec((rb, dout), lambda i: (i, 0)),
        out_shape=jax.ShapeDtypeStruct((n, dout), jnp.float32),
    )(x, w, b.reshape(1, dout))


def _tc_mlp(h, agg, res, w1, b1, lg, lb, w2, b2):
    add_res = res is not None
    row_specs = [pl.BlockSpec((_RB, H), lambda i: (i, 0))] * (3 if add_res else 2)
    args = ([h, agg, res] if add_res else [h, agg]) + [
        w1, b1.reshape(1, -1), lg.reshape(1, -1), lb.reshape(1, -1),
        w2, b2.reshape(1, -1)]
    return pl.pallas_call(
        functools.partial(_mlp_body, add_res=add_res),
        grid=(N // _RB,),
        in_specs=row_specs + [
            pl.BlockSpec((H, 2 * H), lambda i: (0, 0)),
            pl.BlockSpec((1, 2 * H), lambda i: (0, 0)),
            pl.BlockSpec((1, 2 * H), lambda i: (0, 0)),
            pl.BlockSpec((1, 2 * H), lambda i: (0, 0)),
            pl.BlockSpec((2 * H, H), lambda i: (0, 0)),
            pl.BlockSpec((1, H), lambda i: (0, 0)),
        ],
        out_specs=pl.BlockSpec((_RB, H), lambda i: (i, 0)),
        out_shape=jax.ShapeDtypeStruct((N, H), jnp.float32),
    )(*args)


def _tc_prenorm(h, g, b):
    return pl.pallas_call(
        _prenorm_body,
        grid=(N // _RB,),
        in_specs=[
            pl.BlockSpec((_RB, H), lambda i: (i, 0)),
            pl.BlockSpec((1, H), lambda i: (0, 0)),
            pl.BlockSpec((1, H), lambda i: (0, 0)),
        ],
        out_specs=pl.BlockSpec((_RB, H), lambda i: (i, 0)),
        out_shape=jax.ShapeDtypeStruct((N, H), jnp.float32),
    )(h, g.reshape(1, H), b.reshape(1, H))


def _tc_final(h, g, b, batch2d, lw, lb):
    return pl.pallas_call(
        _final_body,
        grid=(N // _RB,),
        in_specs=[
            pl.BlockSpec((_RB, H), lambda i: (i, 0)),
            pl.BlockSpec((1, H), lambda i: (0, 0)),
            pl.BlockSpec((1, H), lambda i: (0, 0)),
            pl.BlockSpec((_RB, 1), lambda i: (i, 0)),
            pl.BlockSpec((H, 1), lambda i: (0, 0)),
            pl.BlockSpec((1, 1), lambda i: (0, 0)),
        ],
        out_specs=[
            pl.BlockSpec((_RB, H), lambda i: (i, 0)),
            pl.BlockSpec((NB, H), lambda i: (0, 0)),
            pl.BlockSpec((1, NB), lambda i: (0, 0)),
        ],
        out_shape=[
            jax.ShapeDtypeStruct((N, H), jnp.float32),
            jax.ShapeDtypeStruct((NB, H), jnp.float32),
            jax.ShapeDtypeStruct((1, NB), jnp.float32),
        ],
        scratch_shapes=[
            pltpu.VMEM((NB, H), jnp.float32),
            pltpu.VMEM((1, NB), jnp.float32),
        ],
    )(h, g.reshape(1, H), b.reshape(1, H), batch2d, lw, lb.reshape(1, 1))


def kernel(x, edge_index, edge_attr, batch, node_W, node_b, edge_W, edge_b,
           t, W1, b1, ln_g, ln_b, W2, b2, norm_g, norm_b, lin_W, lin_b):
    src, dst = edge_index[0], edge_index[1]
    perm = jnp.argsort(dst)
    dst_s = dst[perm]
    src_s = src[perm]

    base = jnp.arange(1, NW, dtype=jnp.int32) * (E // NW)
    fmid = jnp.searchsorted(dst_s, dst_s[base - 1], side="right").astype(jnp.int32)
    f = jnp.concatenate([jnp.zeros((1,), jnp.int32), fmid,
                         jnp.full((1,), E, jnp.int32)])
    gmid = jnp.where(fmid < E, dst_s[jnp.minimum(fmid, E - 1)], N)
    g = jnp.concatenate([jnp.zeros((1,), jnp.int32), gmid,
                         jnp.full((1,), N, jnp.int32)])
    meta = jnp.zeros((NW, 16), jnp.int32)
    meta = meta.at[:, 0].set(f[:-1]).at[:, 1].set(f[1:])
    meta = meta.at[:, 2].set(g[:-1]).at[:, 3].set(g[1:])

    h = _tc_encode(x, node_W, node_b)
    ea = _tc_encode(edge_attr, edge_W, edge_b)

    for l in range(L):
        z = h if l == 0 else _tc_prenorm(h, norm_g[l], norm_b[l])
        tv = jnp.full((16,), t[l], jnp.float32)
        agg = _sc_agg(z, ea, src_s, dst_s, perm, meta, tv)[:N]
        h = _tc_mlp(z, agg, None if l == 0 else h,
                    W1[l], b1[l], ln_g[l], ln_b[l], W2[l], b2[l])

    emb, ge, svb = _tc_final(h, norm_g[0], norm_b[0],
                             batch.reshape(N, 1), lin_W, lin_b)
    return (svb, ge, emb)
